# baseline (speedup 1.0000x reference)
import functools

import jax
import jax.numpy as jnp
from jax import lax
from jax.experimental import pallas as pl
from jax.experimental.pallas import tpu as pltpu
from jax.experimental.pallas import tpu_sc as plsc

D = 32
B = 16384
NC = 2
NS = 16
NW = NC * NS
BPW = B // NW
GPW = BPW // 4

_mesh = plsc.VectorSubcoreMesh(core_axis_name="c", subcore_axis_name="s")


@functools.partial(
    pl.kernel,
    mesh=_mesh,
    out_type=jax.ShapeDtypeStruct((B // 4, 128), jnp.float32),
    scratch_types=[
        pltpu.VMEM((BPW,), jnp.int32),
        pltpu.VMEM((BPW,), jnp.int32),
        pltpu.VMEM((BPW, 128), jnp.float32),
        pltpu.VMEM((GPW, 128), jnp.float32),
        pltpu.VMEM((BPW * 3,), jnp.float32),
        pltpu.VMEM((128,), jnp.float32),
        pltpu.SemaphoreType.DMA,
    ],
    compiler_params=pltpu.CompilerParams(use_tc_tiling_on_sc=True),
)
def _sc_embed(ids_hbm, coords_hbm, table_hbm, wtb_hbm, out_hbm,
              ids_v, row_v, rows_v, out_v, coords_v, wtb_v, sem):
    wid = lax.axis_index("s") * NC + lax.axis_index("c")
    base = wid * BPW

    pltpu.sync_copy(ids_hbm.at[pl.ds(base, BPW)], ids_v)

    def rowids(k, _):
        v = ids_v[pl.ds(k * 16, 16)]
        row_v[pl.ds(k * 16, 16)] = lax.shift_right_logical(v, 2)
        return 0

    lax.fori_loop(0, BPW // 16, rowids, 0)
    gather = pltpu.async_copy(table_hbm.at[row_v], rows_v, sem)

    pltpu.sync_copy(coords_hbm.at[pl.ds(base * 3, BPW * 3)], coords_v)
    pltpu.sync_copy(wtb_hbm, wtb_v)

    w0_lo = wtb_v[pl.ds(0, 16)]
    w0_hi = wtb_v[pl.ds(16, 16)]
    w1_lo = wtb_v[pl.ds(32, 16)]
    w1_hi = wtb_v[pl.ds(48, 16)]
    w2_lo = wtb_v[pl.ds(64, 16)]
    w2_hi = wtb_v[pl.ds(80, 16)]
    b_lo = wtb_v[pl.ds(96, 16)]
    b_hi = wtb_v[pl.ds(112, 16)]

    gather.wait()

    def body(g, _):
        cbase = g * 48
        cv = (coords_v[pl.ds(cbase, 16)],
              coords_v[pl.ds(cbase + 16, 16)],
              coords_v[pl.ds(cbase + 32, 16)])
        q32 = lax.shift_left(ids_v[pl.ds(g * 16, 16)] & 3, 5)
        for j in range(16):
            i = g * 16 + j
            k = 3 * j
            c0 = cv[k // 16][k % 16]
            c1 = cv[(k + 1) // 16][(k + 1) % 16]
            c2 = cv[(k + 2) // 16][(k + 2) % 16]
            q = pl.multiple_of(q32[j], 32)
            e_lo = rows_v[i, pl.ds(q, 16)]
            e_hi = rows_v[i, pl.ds(q + 16, 16)]
            p_lo = e_lo + c0 * w0_lo + c1 * w1_lo + c2 * w2_lo + b_lo
            p_hi = e_hi + c0 * w0_hi + c1 * w1_hi + c2 * w2_hi + b_hi
            orow = g * 4 + j // 4
            ocol = (j % 4) * 32
            out_v[orow, pl.ds(ocol, 16)] = p_lo
            out_v[orow, pl.ds(ocol + 16, 16)] = p_hi
        return 0

    lax.fori_loop(0, BPW // 16, body, 0)

    pltpu.sync_copy(out_v, out_hbm.at[pl.ds(wid * GPW, GPW)])


def kernel(region_ids, spatial_coords, table, W, b):
    ids = region_ids.astype(jnp.int32)
    coords_flat = spatial_coords.reshape(-1)
    table128 = table.reshape(table.shape[0] // 4, 128)
    wtb = jnp.concatenate([W[:, 0], W[:, 1], W[:, 2], b], axis=0)
    out = _sc_embed(ids, coords_flat, table128, wtb)
    return out.reshape(B, D)


# speedup vs baseline: 3.2552x; 3.2552x over previous
import functools

import jax
import jax.numpy as jnp
from jax import lax
from jax.experimental import pallas as pl
from jax.experimental.pallas import tpu as pltpu
from jax.experimental.pallas import tpu_sc as plsc

D = 32
B = 16384
NC = 2
NS = 16
NW = NC * NS
BPW = B // NW
G = 8
NG = BPW // G
RING = 2

_mesh = plsc.VectorSubcoreMesh(core_axis_name="c", subcore_axis_name="s")


@functools.partial(
    pl.kernel,
    mesh=_mesh,
    out_type=jax.ShapeDtypeStruct((B // 4, 128), jnp.float32),
    scratch_types=[
        pltpu.VMEM((BPW,), jnp.int32),
        pltpu.VMEM((RING * D, G * 128), jnp.float32),
        pltpu.VMEM((BPW // 4, 128), jnp.float32),
        pltpu.VMEM((BPW * 3,), jnp.float32),
        pltpu.VMEM((128,), jnp.float32),
        [pltpu.SemaphoreType.DMA] * RING,
    ],
    compiler_params=pltpu.CompilerParams(use_tc_tiling_on_sc=True,
                                         needs_layout_passes=False),
)
def _sc_embed(ids_hbm, coords_hbm, table_hbm, wtb_hbm, out_hbm,
              ids_v, slots_v, out_v, coords_v, wtb_v, sems):
    wid = lax.axis_index("s") * NC + lax.axis_index("c")
    base = wid * BPW

    pltpu.sync_copy(ids_hbm.at[pl.ds(base, BPW)], ids_v)
    pltpu.sync_copy(coords_hbm.at[pl.ds(base * 3, BPW * 3)], coords_v)
    pltpu.sync_copy(wtb_hbm, wtb_v)

    w0 = (wtb_v[pl.ds(0, 16)], wtb_v[pl.ds(16, 16)])
    w1 = (wtb_v[pl.ds(32, 16)], wtb_v[pl.ds(48, 16)])
    w2 = (wtb_v[pl.ds(64, 16)], wtb_v[pl.ds(80, 16)])
    bb = (wtb_v[pl.ds(96, 16)], wtb_v[pl.ds(112, 16)])
    iota = lax.iota(jnp.int32, 16)

    def issue(blk, half, s):
        idv = ids_v[pl.ds(blk * 16, 16)]
        basev = lax.shift_left(lax.shift_right_logical(idv, 7), 7)
        for j in range(G):
            lb = pl.multiple_of(basev[half * G + j], 128)
            pltpu.async_copy(
                table_hbm.at[pl.ds(0, D), pl.ds(lb, 128)],
                slots_v.at[pl.ds(s * D, D), pl.ds(j * 128, 128)],
                sems[s])

    for k in range(RING):
        issue(0, k, k)

    def body(gg, _):
        idv = ids_v[pl.ds(gg * 16, 16)]
        remv = idv & 127
        cbase = gg * 48
        cv = (coords_v[pl.ds(cbase, 16)],
              coords_v[pl.ds(cbase + 16, 16)],
              coords_v[pl.ds(cbase + 32, 16)])
        for s in range(RING):
            g = gg * RING + s
            pltpu.make_async_copy(
                table_hbm.at[pl.ds(0, D), pl.ds(0, G * 128)],
                slots_v.at[pl.ds(s * D, D)],
                sems[s]).wait()

            rlo = iota + (s * D)
            rhi = rlo + 16
            for j in range(G):
                jj = s * G + j
                k = 3 * jj
                c0 = cv[k // 16][k % 16]
                c1 = cv[(k + 1) // 16][(k + 1) % 16]
                c2 = cv[(k + 2) // 16][(k + 2) % 16]
                col = jnp.full((16,), remv[jj] + j * 128, jnp.int32)
                e_lo = plsc.load_gather(slots_v, [rlo, col])
                e_hi = plsc.load_gather(slots_v, [rhi, col])
                p_lo = e_lo + c0 * w0[0] + c1 * w1[0] + c2 * w2[0] + bb[0]
                p_hi = e_hi + c0 * w0[1] + c1 * w1[1] + c2 * w2[1] + bb[1]
                orow = g * 2 + j // 4
                ocol = (j % 4) * 32
                out_v[orow, pl.ds(ocol, 16)] = p_lo
                out_v[orow, pl.ds(ocol + 16, 16)] = p_hi

            @pl.when(g + RING < NG)
            def _():
                issue(gg + 1, s, s)
        return 0

    lax.fori_loop(0, NG // RING, body, 0)

    pltpu.sync_copy(out_v, out_hbm.at[pl.ds(wid * (BPW // 4), BPW // 4)])


def kernel(region_ids, spatial_coords, table, W, b):
    ids = region_ids.astype(jnp.int32)
    coords_flat = spatial_coords.reshape(-1)
    table_t = table.T
    wtb = jnp.concatenate([W[:, 0], W[:, 1], W[:, 2], b], axis=0)
    out = _sc_embed(ids, coords_flat, table_t, wtb)
    return out.reshape(B, D)


# speedup vs baseline: 3.5062x; 1.0771x over previous
import functools

import jax
import jax.numpy as jnp
from jax import lax
from jax.experimental import pallas as pl
from jax.experimental.pallas import tpu as pltpu
from jax.experimental.pallas import tpu_sc as plsc

D = 32
B = 16384
NC = 2
NS = 16
NW = NC * NS
BPW = B // NW
G = 4
NG = BPW // G
RING = 4

_mesh = plsc.VectorSubcoreMesh(core_axis_name="c", subcore_axis_name="s")


@functools.partial(
    pl.kernel,
    mesh=_mesh,
    out_type=jax.ShapeDtypeStruct((B // 4, 128), jnp.float32),
    scratch_types=[
        pltpu.VMEM((BPW,), jnp.int32),
        pltpu.VMEM((RING * D, G * 128), jnp.float32),
        pltpu.VMEM((BPW // 4, 128), jnp.float32),
        pltpu.VMEM((BPW * 3,), jnp.float32),
        pltpu.VMEM((128,), jnp.float32),
        [pltpu.SemaphoreType.DMA] * RING,
    ],
    compiler_params=pltpu.CompilerParams(use_tc_tiling_on_sc=True,
                                         needs_layout_passes=False),
)
def _sc_embed(ids_hbm, coords_hbm, table_hbm, wtb_hbm, out_hbm,
              ids_v, slots_v, out_v, coords_v, wtb_v, sems):
    wid = lax.axis_index("s") * NC + lax.axis_index("c")
    base = wid * BPW

    pltpu.sync_copy(ids_hbm.at[pl.ds(base, BPW)], ids_v)
    pltpu.sync_copy(coords_hbm.at[pl.ds(base * 3, BPW * 3)], coords_v)
    pltpu.sync_copy(wtb_hbm, wtb_v)

    w0 = (wtb_v[pl.ds(0, 16)], wtb_v[pl.ds(16, 16)])
    w1 = (wtb_v[pl.ds(32, 16)], wtb_v[pl.ds(48, 16)])
    w2 = (wtb_v[pl.ds(64, 16)], wtb_v[pl.ds(80, 16)])
    bb = (wtb_v[pl.ds(96, 16)], wtb_v[pl.ds(112, 16)])
    iota = lax.iota(jnp.int32, 16)

    def issue(blk, half, s):
        idv = ids_v[pl.ds(blk * 16, 16)]
        basev = lax.shift_left(lax.shift_right_logical(idv, 7), 7)
        for j in range(G):
            lb = pl.multiple_of(basev[half * G + j], 128)
            pltpu.async_copy(
                table_hbm.at[pl.ds(0, D), pl.ds(lb, 128)],
                slots_v.at[pl.ds(s * D, D), pl.ds(j * 128, 128)],
                sems[s])

    for k in range(RING):
        issue(0, k, k)

    def body(gg, _):
        idv = ids_v[pl.ds(gg * 16, 16)]
        remv = idv & 127
        cbase = gg * 48
        cv = (coords_v[pl.ds(cbase, 16)],
              coords_v[pl.ds(cbase + 16, 16)],
              coords_v[pl.ds(cbase + 32, 16)])
        for s in range(RING):
            g = gg * RING + s
            pltpu.make_async_copy(
                table_hbm.at[pl.ds(0, D), pl.ds(0, G * 128)],
                slots_v.at[pl.ds(s * D, D)],
                sems[s]).wait()

            rlo = iota + (s * D)
            rhi = rlo + 16
            for j in range(G):
                jj = s * G + j
                k = 3 * jj
                c0 = cv[k // 16][k % 16]
                c1 = cv[(k + 1) // 16][(k + 1) % 16]
                c2 = cv[(k + 2) // 16][(k + 2) % 16]
                col = jnp.full((16,), remv[jj] + j * 128, jnp.int32)
                e_lo = plsc.load_gather(slots_v, [rlo, col])
                e_hi = plsc.load_gather(slots_v, [rhi, col])
                p_lo = e_lo + c0 * w0[0] + c1 * w1[0] + c2 * w2[0] + bb[0]
                p_hi = e_hi + c0 * w0[1] + c1 * w1[1] + c2 * w2[1] + bb[1]
                orow = g
                ocol = j * 32
                out_v[orow, pl.ds(ocol, 16)] = p_lo
                out_v[orow, pl.ds(ocol + 16, 16)] = p_hi

            @pl.when(g + RING < NG)
            def _():
                issue(gg + 1, s, s)
        return 0

    lax.fori_loop(0, NG // RING, body, 0)

    pltpu.sync_copy(out_v, out_hbm.at[pl.ds(wid * (BPW // 4), BPW // 4)])


def kernel(region_ids, spatial_coords, table, W, b):
    ids = region_ids.astype(jnp.int32)
    coords_flat = spatial_coords.reshape(-1)
    table_t = table.T
    wtb = jnp.concatenate([W[:, 0], W[:, 1], W[:, 2], b], axis=0)
    out = _sc_embed(ids, coords_flat, table_t, wtb)
    return out.reshape(B, D)
